# Initial kernel scaffold; baseline (speedup 1.0000x reference)
import jax
import jax.numpy as jnp
from jax.experimental import pallas as pl


def kernel(nodes, feat, label, neighbor_0, neighbor_1, neighbor_2, W_self0, W_neigh0, W_self1, W_neigh1, W_pred, b_pred):
    raise NotImplementedError("write your pallas kernel here")


# speedup vs baseline: 2.1897x; 2.1897x over previous
import functools

import jax
import jax.numpy as jnp
from jax import lax
from jax.experimental import pallas as pl
from jax.experimental.pallas import tpu as pltpu
from jax.experimental.pallas import tpu_sc as plsc

IN_DIM = 256
BATCH = 1024
NS0 = 25
NS1 = 10
MID = BATCH * NS1
N2 = MID * NS0
CHUNK = 128
N_ROWS = N2 + MID + BATCH
N_CHUNKS = N_ROWS // CHUNK
C2 = N2 // CHUNK
C1 = C2 + MID // CHUNK
NW = 32
ITERS = -(-N_CHUNKS // NW)

@functools.cache
def _make_sc_gather():
    mesh = plsc.VectorSubcoreMesh(core_axis_name="c", subcore_axis_name="s")

    @functools.partial(
        pl.kernel,
        mesh=mesh,
        out_type=[
            jax.ShapeDtypeStruct((N2, IN_DIM), jnp.float32),
            jax.ShapeDtypeStruct((MID, IN_DIM), jnp.float32),
            jax.ShapeDtypeStruct((BATCH, IN_DIM), jnp.float32),
        ],
        scratch_types=[
            pltpu.VMEM((CHUNK,), jnp.int32),
            pltpu.VMEM((CHUNK, IN_DIM), jnp.float32),
            pltpu.SemaphoreType.DMA,
        ],
    )
    def _sc_gather(feat, idx_all, g2, g1, g0, idx_v, rows_v, sem):
        wid = lax.axis_index("s") * 2 + lax.axis_index("c")

        def body(j, carry):
            c = j * NW + wid

            @pl.when(c < N_CHUNKS)
            def _():
                pltpu.sync_copy(idx_all.at[c], idx_v)
                pltpu.async_copy(feat.at[idx_v], rows_v, sem).wait()

                @pl.when(c < C2)
                def _():
                    pltpu.sync_copy(rows_v, g2.at[pl.ds(c * CHUNK, CHUNK)])

                @pl.when((c >= C2) & (c < C1))
                def _():
                    pltpu.sync_copy(rows_v, g1.at[pl.ds((c - C2) * CHUNK, CHUNK)])

                @pl.when(c >= C1)
                def _():
                    pltpu.sync_copy(rows_v, g0.at[pl.ds((c - C1) * CHUNK, CHUNK)])

            return carry

        lax.fori_loop(0, ITERS, body, 0)

    return _sc_gather


def _tc_mid(g2_ref, g1_ref, ws0_ref, wn0_ref, m1_ref, mh1_ref):
    g2b = g2_ref[...]
    m2 = jnp.sum(g2b, axis=1) * (1.0 / NS0)
    g1b = g1_ref[...]
    h1 = jnp.maximum(
        jnp.dot(g1b, ws0_ref[...], preferred_element_type=jnp.float32)
        + jnp.dot(m2, wn0_ref[...], preferred_element_type=jnp.float32),
        0.0,
    )
    bm = g1b.shape[0]
    rows = lax.broadcasted_iota(jnp.int32, (bm // NS1, bm), 0)
    cols = lax.broadcasted_iota(jnp.int32, (bm // NS1, bm), 1)
    sel = jnp.where(cols // NS1 == rows, 1.0 / NS1, 0.0).astype(jnp.float32)
    m1_ref[...] = jnp.dot(sel, g1b, preferred_element_type=jnp.float32)
    mh1_ref[...] = jnp.dot(sel, h1, preferred_element_type=jnp.float32)


def _tc_head(g0_ref, m1_ref, mh1_ref, lab_ref, ws0_ref, wn0_ref, ws1_ref,
             wn1_ref, wp_ref, bp_ref, loss_ref, pred_ref):
    h0 = jnp.maximum(
        jnp.dot(g0_ref[...], ws0_ref[...], preferred_element_type=jnp.float32)
        + jnp.dot(m1_ref[...], wn0_ref[...], preferred_element_type=jnp.float32),
        0.0,
    )
    out = (
        jnp.dot(h0, ws1_ref[...], preferred_element_type=jnp.float32)
        + jnp.dot(mh1_ref[...], wn1_ref[...], preferred_element_type=jnp.float32)
    )
    nrm = jnp.sqrt(jnp.sum(out * out, axis=1, keepdims=True))
    out = out / jnp.maximum(nrm, 1e-12)
    logits = jnp.dot(out, wp_ref[...], preferred_element_type=jnp.float32) + bp_ref[...]
    mx = jnp.max(logits, axis=1, keepdims=True)
    ex = jnp.exp(logits - mx)
    se = jnp.sum(ex, axis=1, keepdims=True)
    pred_ref[...] = ex / se
    logp = logits - mx - jnp.log(se)
    cols = lax.broadcasted_iota(jnp.int32, logits.shape, 1)
    picked = jnp.sum(jnp.where(cols == lab_ref[...], logp, 0.0), axis=1, keepdims=True)
    loss_ref[...] = jnp.reshape(-jnp.mean(picked), (1, 1))


def kernel(nodes, feat, label, neighbor_0, neighbor_1, neighbor_2,
           W_self0, W_neigh0, W_self1, W_neigh1, W_pred, b_pred):
    del nodes
    idx_all = jnp.concatenate(
        [neighbor_2, neighbor_1, neighbor_0]
    ).astype(jnp.int32).reshape(N_CHUNKS, CHUNK)
    g2, g1, g0 = _make_sc_gather()(feat, idx_all)
    g2_3d = g2.reshape(MID, NS0, IN_DIM)

    BM = 320
    nb = MID // BM
    m1, mh1 = pl.pallas_call(
        _tc_mid,
        grid=(nb,),
        in_specs=[
            pl.BlockSpec((BM, NS0, IN_DIM), lambda i: (i, 0, 0)),
            pl.BlockSpec((BM, IN_DIM), lambda i: (i, 0)),
            pl.BlockSpec((IN_DIM, IN_DIM), lambda i: (0, 0)),
            pl.BlockSpec((IN_DIM, IN_DIM), lambda i: (0, 0)),
        ],
        out_specs=[
            pl.BlockSpec((BM // NS1, IN_DIM), lambda i: (i, 0)),
            pl.BlockSpec((BM // NS1, IN_DIM), lambda i: (i, 0)),
        ],
        out_shape=[
            jax.ShapeDtypeStruct((BATCH, IN_DIM), jnp.float32),
            jax.ShapeDtypeStruct((BATCH, IN_DIM), jnp.float32),
        ],
    )(g2_3d, g1, W_self0, W_neigh0)

    loss2d, pred = pl.pallas_call(
        _tc_head,
        out_shape=[
            jax.ShapeDtypeStruct((1, 1), jnp.float32),
            jax.ShapeDtypeStruct((BATCH, 64), jnp.float32),
        ],
    )(g0, m1, mh1, label, W_self0, W_neigh0, W_self1, W_neigh1,
      W_pred, b_pred.reshape(1, 64))
    return loss2d.reshape(()), pred


# speedup vs baseline: 5.5318x; 2.5263x over previous
import functools

import jax
import jax.numpy as jnp
from jax import lax
from jax.experimental import pallas as pl
from jax.experimental.pallas import tpu as pltpu
from jax.experimental.pallas import tpu_sc as plsc

IN_DIM = 256
BATCH = 1024
NS0 = 25
NS1 = 10
MID = BATCH * NS1
N2 = MID * NS0
CHUNK = 128
C2 = N2 // CHUNK
C2_PER_SC = C2 // 2
OUT_PER_SC = MID // 2
NSUB = 16
ITERS2 = -(-C2_PER_SC // NSUB)
C10 = (MID + BATCH) // CHUNK
C1 = MID // CHUNK
NW = 32
ITERS10 = -(-C10 // NW)
ROWS_PER_SUB = OUT_PER_SC // NSUB


SB = MID // CHUNK
ITERS_SB = -(-SB // NW)


@functools.cache
def _make_sc_gather():
    mesh = plsc.VectorSubcoreMesh(core_axis_name="c", subcore_axis_name="s")

    @functools.partial(
        pl.kernel,
        mesh=mesh,
        out_type=[
            jax.ShapeDtypeStruct((MID, IN_DIM), jnp.float32),
            jax.ShapeDtypeStruct((MID, IN_DIM), jnp.float32),
            jax.ShapeDtypeStruct((BATCH, IN_DIM), jnp.float32),
        ],
        scratch_types=[
            pltpu.VMEM((CHUNK,), jnp.int32),
            pltpu.VMEM((CHUNK, IN_DIM), jnp.float32),
            pltpu.SemaphoreType.DMA,
        ],
    )
    def _sc_gather(feat, idx2t, idx10, s2, g1, g0, idx_v, acc, sem):
        core = lax.axis_index("c")
        sub = lax.axis_index("s")
        wid = sub * 2 + core

        def body_sb(j, carry):
            sb = j * NW + wid

            @pl.when(sb < SB)
            def _():
                pltpu.sync_copy(idx2t.at[0, sb], idx_v)
                pltpu.async_copy(feat.at[idx_v], acc, sem).wait()

                def body_k(k, c2):
                    pltpu.sync_copy(idx2t.at[k, sb], idx_v)
                    pltpu.async_copy(feat.at[idx_v], acc, sem, add=True).wait()
                    return c2

                lax.fori_loop(1, NS0, body_k, 0)
                pltpu.sync_copy(acc, s2.at[pl.ds(sb * CHUNK, CHUNK)])

            return carry

        lax.fori_loop(0, ITERS_SB, body_sb, 0)

        def body10(j, carry):
            c = j * NW + wid

            @pl.when(c < C10)
            def _():
                pltpu.sync_copy(idx10.at[c], idx_v)
                pltpu.async_copy(feat.at[idx_v], acc, sem).wait()

                @pl.when(c < C1)
                def _():
                    pltpu.sync_copy(acc, g1.at[pl.ds(c * CHUNK, CHUNK)])

                @pl.when(c >= C1)
                def _():
                    pltpu.sync_copy(acc, g0.at[pl.ds((c - C1) * CHUNK, CHUNK)])

            return carry

        lax.fori_loop(0, ITERS10, body10, 0)

    return _sc_gather


def _tc_mid(s2_ref, g1_ref, ws0_ref, wn0_ref, m1_ref, mh1_ref):
    m2 = s2_ref[...] * (1.0 / NS0)
    g1b = g1_ref[...]
    h1 = jnp.maximum(
        jnp.dot(g1b, ws0_ref[...], preferred_element_type=jnp.float32)
        + jnp.dot(m2, wn0_ref[...], preferred_element_type=jnp.float32),
        0.0,
    )
    bm = g1b.shape[0]
    rows = lax.broadcasted_iota(jnp.int32, (bm // NS1, bm), 0)
    cols = lax.broadcasted_iota(jnp.int32, (bm // NS1, bm), 1)
    sel = jnp.where(cols // NS1 == rows, 1.0 / NS1, 0.0).astype(jnp.float32)
    m1_ref[...] = jnp.dot(sel, g1b, preferred_element_type=jnp.float32)
    mh1_ref[...] = jnp.dot(sel, h1, preferred_element_type=jnp.float32)


def _tc_head(g0_ref, m1_ref, mh1_ref, lab_ref, ws0_ref, wn0_ref, ws1_ref,
             wn1_ref, wp_ref, bp_ref, loss_ref, pred_ref):
    h0 = jnp.maximum(
        jnp.dot(g0_ref[...], ws0_ref[...], preferred_element_type=jnp.float32)
        + jnp.dot(m1_ref[...], wn0_ref[...], preferred_element_type=jnp.float32),
        0.0,
    )
    out = (
        jnp.dot(h0, ws1_ref[...], preferred_element_type=jnp.float32)
        + jnp.dot(mh1_ref[...], wn1_ref[...], preferred_element_type=jnp.float32)
    )
    nrm = jnp.sqrt(jnp.sum(out * out, axis=1, keepdims=True))
    out = out / jnp.maximum(nrm, 1e-12)
    logits = jnp.dot(out, wp_ref[...], preferred_element_type=jnp.float32) + bp_ref[...]
    mx = jnp.max(logits, axis=1, keepdims=True)
    ex = jnp.exp(logits - mx)
    se = jnp.sum(ex, axis=1, keepdims=True)
    pred_ref[...] = ex / se
    logp = logits - mx - jnp.log(se)
    cols = lax.broadcasted_iota(jnp.int32, logits.shape, 1)
    picked = jnp.sum(jnp.where(cols == lab_ref[...], logp, 0.0), axis=1, keepdims=True)
    loss_ref[...] = jnp.reshape(-jnp.mean(picked), (1, 1))


def kernel(nodes, feat, label, neighbor_0, neighbor_1, neighbor_2,
           W_self0, W_neigh0, W_self1, W_neigh1, W_pred, b_pred):
    del nodes
    idx2t = (
        neighbor_2.astype(jnp.int32)
        .reshape(MID, NS0)
        .transpose(1, 0)
        .reshape(NS0, SB, CHUNK)
    )
    idx10 = jnp.concatenate(
        [neighbor_1, neighbor_0]
    ).astype(jnp.int32).reshape(C10, CHUNK)

    s2, g1, g0 = _make_sc_gather()(feat, idx2t, idx10)

    BM = 320
    nb = MID // BM
    m1, mh1 = pl.pallas_call(
        _tc_mid,
        grid=(nb,),
        in_specs=[
            pl.BlockSpec((BM, IN_DIM), lambda i: (i, 0)),
            pl.BlockSpec((BM, IN_DIM), lambda i: (i, 0)),
            pl.BlockSpec((IN_DIM, IN_DIM), lambda i: (0, 0)),
            pl.BlockSpec((IN_DIM, IN_DIM), lambda i: (0, 0)),
        ],
        out_specs=[
            pl.BlockSpec((BM // NS1, IN_DIM), lambda i: (i, 0)),
            pl.BlockSpec((BM // NS1, IN_DIM), lambda i: (i, 0)),
        ],
        out_shape=[
            jax.ShapeDtypeStruct((BATCH, IN_DIM), jnp.float32),
            jax.ShapeDtypeStruct((BATCH, IN_DIM), jnp.float32),
        ],
    )(s2, g1, W_self0, W_neigh0)

    loss2d, pred = pl.pallas_call(
        _tc_head,
        out_shape=[
            jax.ShapeDtypeStruct((1, 1), jnp.float32),
            jax.ShapeDtypeStruct((BATCH, 64), jnp.float32),
        ],
    )(g0, m1, mh1, label, W_self0, W_neigh0, W_self1, W_neigh1,
      W_pred, b_pred.reshape(1, 64))
    return loss2d.reshape(()), pred


# speedup vs baseline: 8.2823x; 1.4972x over previous
import functools

import jax
import jax.numpy as jnp
from jax import lax
from jax.experimental import pallas as pl
from jax.experimental.pallas import tpu as pltpu
from jax.experimental.pallas import tpu_sc as plsc

IN_DIM = 256
BATCH = 1024
NS0 = 25
NS1 = 10
MID = BATCH * NS1
N2 = MID * NS0
CHUNK = 128
C2 = N2 // CHUNK
C2_PER_SC = C2 // 2
OUT_PER_SC = MID // 2
NSUB = 16
ITERS2 = -(-C2_PER_SC // NSUB)
C10 = (MID + BATCH) // CHUNK
C1 = MID // CHUNK
NW = 32
ITERS10 = -(-C10 // NW)
ROWS_PER_SUB = OUT_PER_SC // NSUB


SBR = 64
SB = MID // SBR
SB_PER_W = SB // NW
CH10 = 64
C10B = (MID + BATCH) // CH10
C1B = MID // CH10
SLOTS10 = -(-C10B // NW)


@functools.cache
def _make_sc_gather():
    mesh = plsc.VectorSubcoreMesh(core_axis_name="c", subcore_axis_name="s")

    @functools.partial(
        pl.kernel,
        mesh=mesh,
        out_type=[
            jax.ShapeDtypeStruct((MID, IN_DIM), jnp.float32),
            jax.ShapeDtypeStruct((MID, IN_DIM), jnp.float32),
            jax.ShapeDtypeStruct((BATCH, IN_DIM), jnp.float32),
        ],
        scratch_types=[
            pltpu.VMEM((NS0, SBR), jnp.int32),
            pltpu.VMEM((SBR, IN_DIM), jnp.float32),
            pltpu.VMEM((SBR, IN_DIM), jnp.float32),
            pltpu.SemaphoreType.DMA,
            pltpu.SemaphoreType.DMA,
        ],
    )
    def _sc_gather(feat, idx2t, idx10, s2, g1, g0,
                   idx_sb, acc_a, acc_b, sem_a, sem_b):
        core = lax.axis_index("c")
        sub = lax.axis_index("s")
        wid = sub * 2 + core

        bufs = ((acc_a, sem_a), (acc_b, sem_b))

        def do_out10(c, buf):
            @pl.when(c < C1B)
            def _():
                pltpu.sync_copy(buf, g1.at[pl.ds(c * CH10, CH10)])

            @pl.when(c >= C1B)
            def _():
                pltpu.sync_copy(buf, g0.at[pl.ds((c - C1B) * CH10, CH10)])

        for t in range(SLOTS10):
            c = t * NW + wid
            buf, sem = bufs[t % 2]

            @pl.when(c < C10B)
            def _(c=c, buf=buf, sem=sem, t=t):
                pltpu.sync_copy(idx10.at[c], idx_sb.at[t % 2])
                pltpu.async_copy(feat.at[idx_sb.at[t % 2]], buf, sem)

            if t >= 1:
                pc = (t - 1) * NW + wid
                pbuf, psem = bufs[(t - 1) % 2]

                @pl.when(pc < C10B)
                def _(pc=pc, pbuf=pbuf, psem=psem):
                    pltpu.make_async_copy(feat.at[idx_sb.at[0]], pbuf, psem).wait()
                    do_out10(pc, pbuf)

        lc = (SLOTS10 - 1) * NW + wid
        lbuf, lsem = bufs[(SLOTS10 - 1) % 2]

        @pl.when(lc < C10B)
        def _():
            pltpu.make_async_copy(feat.at[idx_sb.at[0]], lbuf, lsem).wait()
            do_out10(lc, lbuf)

        for jj in range(SB_PER_W):
            sb = jj * NW + wid
            acc, sem = bufs[jj % 2]
            pltpu.sync_copy(idx2t.at[sb], idx_sb)
            pltpu.async_copy(feat.at[idx_sb.at[0]], acc, sem).wait()

            def fire(k, c2, acc=acc, sem=sem):
                pltpu.async_copy(feat.at[idx_sb.at[k]], acc, sem, add=True)
                return c2

            lax.fori_loop(1, NS0, fire, 0)

            def drain(k, c2, acc=acc, sem=sem):
                pltpu.make_async_copy(feat.at[idx_sb.at[0]], acc, sem).wait()
                return c2

            lax.fori_loop(1, NS0, drain, 0)
            pltpu.sync_copy(acc, s2.at[pl.ds(sb * SBR, SBR)])

    return _sc_gather


def _tc_mid(s2_ref, g1_ref, ws0_ref, wn0_ref, m1_ref, mh1_ref):
    m2 = s2_ref[...] * (1.0 / NS0)
    g1b = g1_ref[...]
    h1 = jnp.maximum(
        jnp.dot(g1b, ws0_ref[...], preferred_element_type=jnp.float32)
        + jnp.dot(m2, wn0_ref[...], preferred_element_type=jnp.float32),
        0.0,
    )
    bm = g1b.shape[0]
    rows = lax.broadcasted_iota(jnp.int32, (bm // NS1, bm), 0)
    cols = lax.broadcasted_iota(jnp.int32, (bm // NS1, bm), 1)
    sel = jnp.where(cols // NS1 == rows, 1.0 / NS1, 0.0).astype(jnp.float32)
    m1_ref[...] = jnp.dot(sel, g1b, preferred_element_type=jnp.float32)
    mh1_ref[...] = jnp.dot(sel, h1, preferred_element_type=jnp.float32)


def _tc_head(g0_ref, m1_ref, mh1_ref, lab_ref, ws0_ref, wn0_ref, ws1_ref,
             wn1_ref, wp_ref, bp_ref, loss_ref, pred_ref):
    h0 = jnp.maximum(
        jnp.dot(g0_ref[...], ws0_ref[...], preferred_element_type=jnp.float32)
        + jnp.dot(m1_ref[...], wn0_ref[...], preferred_element_type=jnp.float32),
        0.0,
    )
    out = (
        jnp.dot(h0, ws1_ref[...], preferred_element_type=jnp.float32)
        + jnp.dot(mh1_ref[...], wn1_ref[...], preferred_element_type=jnp.float32)
    )
    nrm = jnp.sqrt(jnp.sum(out * out, axis=1, keepdims=True))
    out = out / jnp.maximum(nrm, 1e-12)
    logits = jnp.dot(out, wp_ref[...], preferred_element_type=jnp.float32) + bp_ref[...]
    mx = jnp.max(logits, axis=1, keepdims=True)
    ex = jnp.exp(logits - mx)
    se = jnp.sum(ex, axis=1, keepdims=True)
    pred_ref[...] = ex / se
    logp = logits - mx - jnp.log(se)
    cols = lax.broadcasted_iota(jnp.int32, logits.shape, 1)
    picked = jnp.sum(jnp.where(cols == lab_ref[...], logp, 0.0), axis=1, keepdims=True)
    loss_ref[...] = jnp.reshape(-jnp.mean(picked), (1, 1))


def kernel(nodes, feat, label, neighbor_0, neighbor_1, neighbor_2,
           W_self0, W_neigh0, W_self1, W_neigh1, W_pred, b_pred):
    del nodes
    idx2t = (
        neighbor_2.astype(jnp.int32)
        .reshape(SB, SBR, NS0)
        .transpose(0, 2, 1)
    )
    idx10 = jnp.concatenate(
        [neighbor_1, neighbor_0]
    ).astype(jnp.int32).reshape(C10B, CH10)

    s2, g1, g0 = _make_sc_gather()(feat, idx2t, idx10)

    BM = 320
    nb = MID // BM
    m1, mh1 = pl.pallas_call(
        _tc_mid,
        grid=(nb,),
        in_specs=[
            pl.BlockSpec((BM, IN_DIM), lambda i: (i, 0)),
            pl.BlockSpec((BM, IN_DIM), lambda i: (i, 0)),
            pl.BlockSpec((IN_DIM, IN_DIM), lambda i: (0, 0)),
            pl.BlockSpec((IN_DIM, IN_DIM), lambda i: (0, 0)),
        ],
        out_specs=[
            pl.BlockSpec((BM // NS1, IN_DIM), lambda i: (i, 0)),
            pl.BlockSpec((BM // NS1, IN_DIM), lambda i: (i, 0)),
        ],
        out_shape=[
            jax.ShapeDtypeStruct((BATCH, IN_DIM), jnp.float32),
            jax.ShapeDtypeStruct((BATCH, IN_DIM), jnp.float32),
        ],
    )(s2, g1, W_self0, W_neigh0)

    loss2d, pred = pl.pallas_call(
        _tc_head,
        out_shape=[
            jax.ShapeDtypeStruct((1, 1), jnp.float32),
            jax.ShapeDtypeStruct((BATCH, 64), jnp.float32),
        ],
    )(g0, m1, mh1, label, W_self0, W_neigh0, W_self1, W_neigh1,
      W_pred, b_pred.reshape(1, 64))
    return loss2d.reshape(()), pred


# speedup vs baseline: 8.7570x; 1.0573x over previous
import functools

import jax
import jax.numpy as jnp
from jax import lax
from jax.experimental import pallas as pl
from jax.experimental.pallas import tpu as pltpu
from jax.experimental.pallas import tpu_sc as plsc

IN_DIM = 256
BATCH = 1024
NS0 = 25
NS1 = 10
MID = BATCH * NS1
N2 = MID * NS0
CHUNK = 128
C2 = N2 // CHUNK
C2_PER_SC = C2 // 2
OUT_PER_SC = MID // 2
NSUB = 16
ITERS2 = -(-C2_PER_SC // NSUB)
C10 = (MID + BATCH) // CHUNK
C1 = MID // CHUNK
NW = 32
ITERS10 = -(-C10 // NW)
ROWS_PER_SUB = OUT_PER_SC // NSUB


SBR = 64
SB = MID // SBR
SB_PER_W = SB // NW
CH10 = 64
C10B = (MID + BATCH) // CH10
C1B = MID // CH10
SLOTS10 = -(-C10B // NW)


@functools.cache
def _make_sc_gather():
    mesh = plsc.VectorSubcoreMesh(core_axis_name="c", subcore_axis_name="s")

    @functools.partial(
        pl.kernel,
        mesh=mesh,
        out_type=[
            jax.ShapeDtypeStruct((MID, IN_DIM), jnp.float32),
            jax.ShapeDtypeStruct((MID, IN_DIM), jnp.float32),
            jax.ShapeDtypeStruct((BATCH, IN_DIM), jnp.float32),
        ],
        scratch_types=[
            pltpu.VMEM((NS0, SBR), jnp.int32),
            pltpu.VMEM((NS0, SBR), jnp.int32),
            pltpu.VMEM((SBR, IN_DIM), jnp.float32),
            pltpu.VMEM((SBR, IN_DIM), jnp.float32),
            pltpu.SemaphoreType.DMA,
            pltpu.SemaphoreType.DMA,
        ],
    )
    def _sc_gather(feat, idx2t, idx10, s2, g1, g0,
                   idx_a, idx_b, acc_a, acc_b, sem_a, sem_b):
        idx_sb = idx_a
        core = lax.axis_index("c")
        sub = lax.axis_index("s")
        wid = sub * 2 + core

        bufs = ((acc_a, sem_a), (acc_b, sem_b))

        def do_out10(c, buf):
            @pl.when(c < C1B)
            def _():
                pltpu.sync_copy(buf, g1.at[pl.ds(c * CH10, CH10)])

            @pl.when(c >= C1B)
            def _():
                pltpu.sync_copy(buf, g0.at[pl.ds((c - C1B) * CH10, CH10)])

        for t in range(SLOTS10):
            c = t * NW + wid
            buf, sem = bufs[t % 2]

            @pl.when(c < C10B)
            def _(c=c, buf=buf, sem=sem, t=t):
                pltpu.sync_copy(idx10.at[c], idx_sb.at[t % 2])
                pltpu.async_copy(feat.at[idx_sb.at[t % 2]], buf, sem)

            if t >= 1:
                pc = (t - 1) * NW + wid
                pbuf, psem = bufs[(t - 1) % 2]

                @pl.when(pc < C10B)
                def _(pc=pc, pbuf=pbuf, psem=psem):
                    pltpu.make_async_copy(feat.at[idx_sb.at[0]], pbuf, psem).wait()
                    do_out10(pc, pbuf)

        lc = (SLOTS10 - 1) * NW + wid
        lbuf, lsem = bufs[(SLOTS10 - 1) % 2]

        @pl.when(lc < C10B)
        def _():
            pltpu.make_async_copy(feat.at[idx_sb.at[0]], lbuf, lsem).wait()
            do_out10(lc, lbuf)

        idxs = (idx_a, idx_b)

        def fires(idxr, acc, sem):
            def fire(k, c2):
                pltpu.async_copy(feat.at[idxr.at[k]], acc, sem, add=True)
                return c2

            lax.fori_loop(1, NS0, fire, 0)

        def drains(idxr, acc, sem, sb):
            def drain(k, c2):
                pltpu.make_async_copy(feat.at[idxr.at[0]], acc, sem).wait()
                return c2

            lax.fori_loop(1, NS0, drain, 0)
            pltpu.sync_copy(acc, s2.at[pl.ds(sb * SBR, SBR)])

        for jj in range(SB_PER_W):
            sb = jj * NW + wid
            acc, sem = bufs[jj % 2]
            idxr = idxs[jj % 2]
            pltpu.sync_copy(idx2t.at[sb], idxr)
            k0 = pltpu.async_copy(feat.at[idxr.at[0]], acc, sem)
            if jj >= 1:
                pacc, psem = bufs[(jj - 1) % 2]
                drains(idxs[(jj - 1) % 2], pacc, psem, (jj - 1) * NW + wid)
            k0.wait()
            fires(idxr, acc, sem)

        lj = SB_PER_W - 1
        drains(idxs[lj % 2], bufs[lj % 2][0], bufs[lj % 2][1], lj * NW + wid)

    return _sc_gather


BM = 320
NB = MID // BM
BG = BM // NS1


def _tc_all(s2_ref, g1_ref, g0_ref, lab_ref, ws0_ref, wn0_ref, ws1_ref,
            wn1_ref, wp_ref, bp_ref, loss_ref, pred_ref, m1_s, mh1_s):
    i = pl.program_id(0)
    m2 = s2_ref[...] * (1.0 / NS0)
    g1b = g1_ref[...]
    h1 = jnp.maximum(
        jnp.dot(g1b, ws0_ref[...], preferred_element_type=jnp.float32)
        + jnp.dot(m2, wn0_ref[...], preferred_element_type=jnp.float32),
        0.0,
    )
    rows = lax.broadcasted_iota(jnp.int32, (BG, BM), 0)
    cols = lax.broadcasted_iota(jnp.int32, (BG, BM), 1)
    sel = jnp.where(cols // NS1 == rows, 1.0 / NS1, 0.0).astype(jnp.float32)
    m1_s[pl.ds(i * BG, BG), :] = jnp.dot(sel, g1b, preferred_element_type=jnp.float32)
    mh1_s[pl.ds(i * BG, BG), :] = jnp.dot(sel, h1, preferred_element_type=jnp.float32)

    @pl.when(i == NB - 1)
    def _():
        h0 = jnp.maximum(
            jnp.dot(g0_ref[...], ws0_ref[...], preferred_element_type=jnp.float32)
            + jnp.dot(m1_s[...], wn0_ref[...], preferred_element_type=jnp.float32),
            0.0,
        )
        out = (
            jnp.dot(h0, ws1_ref[...], preferred_element_type=jnp.float32)
            + jnp.dot(mh1_s[...], wn1_ref[...], preferred_element_type=jnp.float32)
        )
        nrm = jnp.sqrt(jnp.sum(out * out, axis=1, keepdims=True))
        out = out / jnp.maximum(nrm, 1e-12)
        logits = (
            jnp.dot(out, wp_ref[...], preferred_element_type=jnp.float32)
            + bp_ref[...]
        )
        mx = jnp.max(logits, axis=1, keepdims=True)
        ex = jnp.exp(logits - mx)
        se = jnp.sum(ex, axis=1, keepdims=True)
        pred_ref[...] = ex / se
        logp = logits - mx - jnp.log(se)
        ocols = lax.broadcasted_iota(jnp.int32, logits.shape, 1)
        picked = jnp.sum(
            jnp.where(ocols == lab_ref[...], logp, 0.0), axis=1, keepdims=True
        )
        loss_ref[...] = jnp.reshape(-jnp.mean(picked), (1, 1))


def kernel(nodes, feat, label, neighbor_0, neighbor_1, neighbor_2,
           W_self0, W_neigh0, W_self1, W_neigh1, W_pred, b_pred):
    del nodes
    idx2t = (
        neighbor_2.astype(jnp.int32)
        .reshape(SB, SBR, NS0)
        .transpose(0, 2, 1)
    )
    idx10 = jnp.concatenate(
        [neighbor_1, neighbor_0]
    ).astype(jnp.int32).reshape(C10B, CH10)

    s2, g1, g0 = _make_sc_gather()(feat, idx2t, idx10)

    loss2d, pred = pl.pallas_call(
        _tc_all,
        grid=(NB,),
        in_specs=[
            pl.BlockSpec((BM, IN_DIM), lambda i: (i, 0)),
            pl.BlockSpec((BM, IN_DIM), lambda i: (i, 0)),
            pl.BlockSpec((BATCH, IN_DIM), lambda i: (0, 0)),
            pl.BlockSpec((BATCH, 1), lambda i: (0, 0)),
            pl.BlockSpec((IN_DIM, IN_DIM), lambda i: (0, 0)),
            pl.BlockSpec((IN_DIM, IN_DIM), lambda i: (0, 0)),
            pl.BlockSpec((IN_DIM, IN_DIM), lambda i: (0, 0)),
            pl.BlockSpec((IN_DIM, IN_DIM), lambda i: (0, 0)),
            pl.BlockSpec((IN_DIM, 64), lambda i: (0, 0)),
            pl.BlockSpec((1, 64), lambda i: (0, 0)),
        ],
        out_specs=[
            pl.BlockSpec((1, 1), lambda i: (0, 0)),
            pl.BlockSpec((BATCH, 64), lambda i: (0, 0)),
        ],
        out_shape=[
            jax.ShapeDtypeStruct((1, 1), jnp.float32),
            jax.ShapeDtypeStruct((BATCH, 64), jnp.float32),
        ],
        scratch_shapes=[
            pltpu.VMEM((BATCH, IN_DIM), jnp.float32),
            pltpu.VMEM((BATCH, IN_DIM), jnp.float32),
        ],
    )(s2, g1, g0, label, W_self0, W_neigh0, W_self1, W_neigh1,
      W_pred, b_pred.reshape(1, 64))
    return loss2d.reshape(()), pred
